# baseline (speedup 1.0000x reference)
import jax
import jax.numpy as jnp
from jax import lax
from jax.experimental import pallas as pl
from jax.experimental.pallas import tpu as pltpu
from jax.experimental.pallas import tpu_sc as plsc

N = 100000
D = 128
S = 64
BLK = 400
NBLK = N // BLK
NW = 32
GRP = BLK // 16
CAP = (NBLK + NW - 1) // NW
WLEN = CAP * BLK


def _wid():
    return lax.axis_index("s") * 2 + lax.axis_index("c")


def _nblk(wid):
    return (NBLK - wid + NW - 1) // NW


def _z_read(mzv):
    zero16 = jnp.zeros((16,), jnp.int32)
    zv = plsc.load_gather(mzv, [zero16])
    invzv = jnp.ones((16,), jnp.float32) / zv
    return invzv


BT = 4096
NB_TC = (N + BT - 1) // BT
NPAD = NB_TC * BT


def _k1_body(x_ref, w_ref, o_ref, zz_ref, zacc):
    i = pl.program_id(0)
    wv = lax.dot_general(w_ref[...], x_ref[...],
                         (((1,), (1,)), ((), ())),
                         preferred_element_type=jnp.float32)
    o_ref[...] = wv[0]
    col = lax.broadcasted_iota(jnp.int32, (1, BT), 1)
    ev = jnp.where(col < N - i * BT, jnp.exp(wv), 0.0)
    z_blk = jnp.sum(ev, axis=1, keepdims=True)

    @pl.when(i == 0)
    def _():
        zacc[...] = jnp.zeros((1, 1), jnp.float32)

    z_new = zacc[...] + z_blk
    zacc[...] = z_new

    @pl.when(i == NB_TC - 1)
    def _():
        zz_ref[...] = z_new


def _fire_drain_wb(w_hbm, b_hbm, wbuf, bbuf, semw, semb, wid, nblk):
    for i in range(CAP):
        @pl.when(i < nblk)
        def _():
            r0 = (wid + i * NW) * BLK
            pltpu.async_copy(w_hbm.at[pl.ds(r0, BLK)],
                             wbuf.at[pl.ds(i * BLK, BLK)], semw)
            pltpu.async_copy(b_hbm.at[pl.ds(r0, BLK)],
                             bbuf.at[pl.ds(i * BLK, BLK)], semb)
    for i in range(CAP):
        @pl.when(i < nblk)
        def _():
            pltpu.make_async_copy(w_hbm.at[pl.ds(0, BLK)],
                                  wbuf.at[pl.ds(0, BLK)], semw).wait()
            pltpu.make_async_copy(b_hbm.at[pl.ds(0, BLK)],
                                  bbuf.at[pl.ds(0, BLK)], semb).wait()


def _k2_body(w_hbm, b_hbm, mz_hbm, dp_hbm, wbuf, bbuf, mzv, denomv,
             semw, semb):
    wid = _wid()
    nblk = _nblk(wid)
    _fire_drain_wb(w_hbm, b_hbm, wbuf, bbuf, semw, semb, wid, nblk)
    pltpu.sync_copy(mz_hbm, mzv)
    invzv = _z_read(mzv)
    zero16 = jnp.zeros((16,), jnp.float32)
    for c in range(S // 16):
        denomv[pl.ds(16 * c, 16)] = zero16

    def grp_body(g, _):
        wvec = wbuf[pl.ds(16 * g, 16)]
        ev = jnp.exp(jnp.exp(wvec) * invzv)
        plsc.addupdate_scatter(denomv, [bbuf[pl.ds(16 * g, 16)]], ev)
        return 0

    lax.fori_loop(0, nblk * GRP, grp_body, 0)
    pltpu.sync_copy(denomv, dp_hbm.at[wid])


def _k3_body(x_hbm, w_hbm, b_hbm, mz_hbm, dp_hbm, acc_hbm,
             xv0, xv1, wbuf, bbuf, mzv, dpv, cinvv, accv,
             sem0, sem1, semw, semb):
    wid = _wid()
    nblk = _nblk(wid)
    pltpu.async_copy(x_hbm.at[pl.ds(wid * BLK, BLK)], xv0, sem0)
    _fire_drain_wb(w_hbm, b_hbm, wbuf, bbuf, semw, semb, wid, nblk)
    pltpu.sync_copy(mz_hbm, mzv)
    invzv = _z_read(mzv)
    pltpu.sync_copy(dp_hbm, dpv)
    one16 = jnp.ones((16,), jnp.float32)
    for c in range(S // 16):
        s = dpv[0, pl.ds(16 * c, 16)]
        for r in range(1, NW):
            s = s + dpv[r, pl.ds(16 * c, 16)]
        cinvv[pl.ds(16 * c, 16)] = one16 / s

    zero16 = jnp.zeros((16,), jnp.float32)

    def zero_body(r, _):
        for j in range(D // 16):
            accv[r, pl.ds(16 * j, 16)] = zero16
        return 0

    lax.fori_loop(0, S, zero_body, 0)

    def blk_body(i, _):
        even = (i % 2) == 0
        nxt = i + 1

        @pl.when(jnp.logical_and(nxt < nblk, even))
        def _():
            pltpu.async_copy(
                x_hbm.at[pl.ds((wid + nxt * NW) * BLK, BLK)], xv1, sem1)

        @pl.when(jnp.logical_and(nxt < nblk, jnp.logical_not(even)))
        def _():
            pltpu.async_copy(
                x_hbm.at[pl.ds((wid + nxt * NW) * BLK, BLK)], xv0, sem0)

        def mk(xv, sem):
            def go():
                pltpu.make_async_copy(x_hbm.at[pl.ds(0, BLK)], xv, sem).wait()

                def grp_body(g, _):
                    wvec = wbuf[pl.ds(i * BLK + 16 * g, 16)]
                    bvec = bbuf[pl.ds(i * BLK + 16 * g, 16)]
                    gv = jnp.exp(wvec) * invzv
                    nw = jnp.exp(gv) * plsc.load_gather(cinvv, [bvec])

                    def uniform():
                        accs = [jnp.zeros((16,), jnp.float32)
                                for _ in range(D // 16)]
                        for l in range(16):
                            r = 16 * g + l
                            sv = jnp.full((16,), nw[l], jnp.float32)
                            for j in range(D // 16):
                                accs[j] = accs[j] + xv[r, pl.ds(16 * j, 16)] * sv
                        bi = bvec[0]
                        for j in range(D // 16):
                            plsc.addupdate(accv.at[bi, pl.ds(16 * j, 16)],
                                           accs[j])

                    def mixed():
                        for l in range(16):
                            bi = bvec[l]
                            sv = jnp.full((16,), nw[l], jnp.float32)
                            r = 16 * g + l
                            for j in range(D // 16):
                                plsc.addupdate(
                                    accv.at[bi, pl.ds(16 * j, 16)],
                                    xv[r, pl.ds(16 * j, 16)] * sv)

                    lax.cond(bvec[0] == bvec[15], uniform, mixed)
                    return 0

                return lax.fori_loop(0, GRP, grp_body, 0)
            return go

        return lax.cond(even, mk(xv0, sem0), mk(xv1, sem1))

    lax.fori_loop(0, nblk, blk_body, 0)
    pltpu.sync_copy(accv, acc_hbm.at[wid])


def _k4_body(a_ref, o_ref):
    o_ref[...] = jnp.sum(a_ref[...], axis=0)


@jax.jit
def kernel(x, batch, W, b):
    f32 = jnp.float32
    i32 = jnp.int32

    k1 = pl.pallas_call(
        _k1_body,
        grid=(NB_TC,),
        in_specs=[
            pl.BlockSpec((BT, D), lambda i: (i, 0)),
            pl.BlockSpec((1, D), lambda i: (0, 0)),
        ],
        out_specs=[
            pl.BlockSpec((BT,), lambda i: (i,)),
            pl.BlockSpec((1, 1), lambda i: (0, 0)),
        ],
        out_shape=[
            jax.ShapeDtypeStruct((NPAD,), f32),
            jax.ShapeDtypeStruct((1, 1), f32),
        ],
        scratch_shapes=[pltpu.VMEM((1, 1), f32)],
    )
    w_arr, z1 = k1(x, W.astype(f32))
    mz = jnp.broadcast_to(z1.reshape(1), (128,))
    batch_i = batch.astype(i32)

    mesh = plsc.VectorSubcoreMesh(core_axis_name="c", subcore_axis_name="s")
    params = pltpu.CompilerParams(needs_layout_passes=False)

    k2 = pl.kernel(
        _k2_body,
        out_type=jax.ShapeDtypeStruct((NW, S), f32),
        mesh=mesh,
        compiler_params=params,
        scratch_types=[
            pltpu.VMEM((WLEN,), f32),
            pltpu.VMEM((WLEN,), i32),
            pltpu.VMEM((128,), f32),
            pltpu.VMEM((S,), f32),
            pltpu.SemaphoreType.DMA,
            pltpu.SemaphoreType.DMA,
        ],
    )
    dpart = k2(w_arr, batch_i, mz)

    k3 = pl.kernel(
        _k3_body,
        out_type=jax.ShapeDtypeStruct((NW, S, D), f32),
        mesh=mesh,
        compiler_params=params,
        scratch_types=[
            pltpu.VMEM((BLK, D), f32),
            pltpu.VMEM((BLK, D), f32),
            pltpu.VMEM((WLEN,), f32),
            pltpu.VMEM((WLEN,), i32),
            pltpu.VMEM((128,), f32),
            pltpu.VMEM((NW, S), f32),
            pltpu.VMEM((S,), f32),
            pltpu.VMEM((S, D), f32),
            pltpu.SemaphoreType.DMA,
            pltpu.SemaphoreType.DMA,
            pltpu.SemaphoreType.DMA,
            pltpu.SemaphoreType.DMA,
        ],
    )
    acc = k3(x, w_arr, batch_i, mz, dpart)

    pooled = pl.pallas_call(
        _k4_body,
        out_shape=jax.ShapeDtypeStruct((S, D), f32),
    )(acc)
    return pooled


# speedup vs baseline: 1.0845x; 1.0845x over previous
import jax
import jax.numpy as jnp
from jax import lax
from jax.experimental import pallas as pl
from jax.experimental.pallas import tpu as pltpu
from jax.experimental.pallas import tpu_sc as plsc

N = 100000
D = 128
S = 64
BLK = 400
NBLK = N // BLK
NW = 32
GRP = BLK // 16
CAP = (NBLK + NW - 1) // NW
WLEN = CAP * BLK


def _wid():
    return lax.axis_index("s") * 2 + lax.axis_index("c")


def _nblk(wid):
    return (NBLK - wid + NW - 1) // NW


def _z_read(mzv):
    zero16 = jnp.zeros((16,), jnp.int32)
    zv = plsc.load_gather(mzv, [zero16])
    invzv = jnp.ones((16,), jnp.float32) / zv
    return invzv


BT = 8192
NB_TC = (N + BT - 1) // BT
NPAD = NB_TC * BT


def _k1_body(x_ref, w_ref, o_ref, zz_ref, zacc):
    i = pl.program_id(0)
    wv = lax.dot_general(w_ref[...], x_ref[...],
                         (((1,), (1,)), ((), ())),
                         preferred_element_type=jnp.float32)
    o_ref[...] = wv[0]
    col = lax.broadcasted_iota(jnp.int32, (1, BT), 1)
    ev = jnp.where(col < N - i * BT, jnp.exp(wv), 0.0)
    z_blk = jnp.sum(ev, axis=1, keepdims=True)

    @pl.when(i == 0)
    def _():
        zacc[...] = jnp.zeros((1, 1), jnp.float32)

    z_new = zacc[...] + z_blk
    zacc[...] = z_new

    @pl.when(i == NB_TC - 1)
    def _():
        zz_ref[...] = z_new


def _fire_drain_wb(w_hbm, b_hbm, wbuf, bbuf, semw, semb, wid, nblk):
    for i in range(CAP):
        @pl.when(i < nblk)
        def _():
            r0 = (wid + i * NW) * BLK
            pltpu.async_copy(w_hbm.at[pl.ds(r0, BLK)],
                             wbuf.at[pl.ds(i * BLK, BLK)], semw)
            pltpu.async_copy(b_hbm.at[pl.ds(r0, BLK)],
                             bbuf.at[pl.ds(i * BLK, BLK)], semb)
    for i in range(CAP):
        @pl.when(i < nblk)
        def _():
            pltpu.make_async_copy(w_hbm.at[pl.ds(0, BLK)],
                                  wbuf.at[pl.ds(0, BLK)], semw).wait()
            pltpu.make_async_copy(b_hbm.at[pl.ds(0, BLK)],
                                  bbuf.at[pl.ds(0, BLK)], semb).wait()


def _k2_body(w_hbm, b_hbm, mz_hbm, dp_hbm, wbuf, bbuf, mzv, denomv,
             semw, semb):
    wid = _wid()
    nblk = _nblk(wid)
    _fire_drain_wb(w_hbm, b_hbm, wbuf, bbuf, semw, semb, wid, nblk)
    pltpu.sync_copy(mz_hbm, mzv)
    invzv = _z_read(mzv)
    zero16 = jnp.zeros((16,), jnp.float32)
    for c in range(S // 16):
        denomv[pl.ds(16 * c, 16)] = zero16

    def grp_body(g, _):
        wvec = wbuf[pl.ds(16 * g, 16)]
        ev = jnp.exp(jnp.exp(wvec) * invzv)
        plsc.addupdate_scatter(denomv, [bbuf[pl.ds(16 * g, 16)]], ev)
        return 0

    lax.fori_loop(0, nblk * GRP, grp_body, 0)
    pltpu.sync_copy(denomv, dp_hbm.at[wid])


def _k3_body(x_hbm, w_hbm, b_hbm, mz_hbm, dp_hbm, acc_hbm,
             xv0, xv1, wbuf, bbuf, mzv, dpv, cinvv, accv,
             sem0, sem1, semw, semb):
    wid = _wid()
    nblk = _nblk(wid)
    pltpu.async_copy(x_hbm.at[pl.ds(wid * BLK, BLK)], xv0, sem0)
    _fire_drain_wb(w_hbm, b_hbm, wbuf, bbuf, semw, semb, wid, nblk)
    pltpu.sync_copy(mz_hbm, mzv)
    invzv = _z_read(mzv)
    pltpu.sync_copy(dp_hbm, dpv)
    one16 = jnp.ones((16,), jnp.float32)
    for c in range(S // 16):
        s = dpv[0, pl.ds(16 * c, 16)]
        for r in range(1, NW):
            s = s + dpv[r, pl.ds(16 * c, 16)]
        cinvv[pl.ds(16 * c, 16)] = one16 / s

    zero16 = jnp.zeros((16,), jnp.float32)

    def zero_body(r, _):
        for j in range(D // 16):
            accv[r, pl.ds(16 * j, 16)] = zero16
        return 0

    lax.fori_loop(0, S, zero_body, 0)

    def blk_body(i, _):
        even = (i % 2) == 0
        nxt = i + 1

        @pl.when(jnp.logical_and(nxt < nblk, even))
        def _():
            pltpu.async_copy(
                x_hbm.at[pl.ds((wid + nxt * NW) * BLK, BLK)], xv1, sem1)

        @pl.when(jnp.logical_and(nxt < nblk, jnp.logical_not(even)))
        def _():
            pltpu.async_copy(
                x_hbm.at[pl.ds((wid + nxt * NW) * BLK, BLK)], xv0, sem0)

        def mk(xv, sem):
            def go():
                pltpu.make_async_copy(x_hbm.at[pl.ds(0, BLK)], xv, sem).wait()

                def grp_body(g, _):
                    wvec = wbuf[pl.ds(i * BLK + 16 * g, 16)]
                    bvec = bbuf[pl.ds(i * BLK + 16 * g, 16)]
                    gv = jnp.exp(wvec) * invzv
                    nw = jnp.exp(gv) * plsc.load_gather(cinvv, [bvec])

                    def uniform():
                        accs = [jnp.zeros((16,), jnp.float32)
                                for _ in range(D // 16)]
                        for l in range(16):
                            r = 16 * g + l
                            sv = jnp.full((16,), nw[l], jnp.float32)
                            for j in range(D // 16):
                                accs[j] = accs[j] + xv[r, pl.ds(16 * j, 16)] * sv
                        bi = bvec[0]
                        for j in range(D // 16):
                            plsc.addupdate(accv.at[bi, pl.ds(16 * j, 16)],
                                           accs[j])

                    def mixed():
                        for l in range(16):
                            bi = bvec[l]
                            sv = jnp.full((16,), nw[l], jnp.float32)
                            r = 16 * g + l
                            for j in range(D // 16):
                                plsc.addupdate(
                                    accv.at[bi, pl.ds(16 * j, 16)],
                                    xv[r, pl.ds(16 * j, 16)] * sv)

                    lax.cond(bvec[0] == bvec[15], uniform, mixed)
                    return 0

                return lax.fori_loop(0, GRP, grp_body, 0)
            return go

        return lax.cond(even, mk(xv0, sem0), mk(xv1, sem1))

    lax.fori_loop(0, nblk, blk_body, 0)
    pltpu.sync_copy(accv, acc_hbm.at[wid])


def _k4_body(a_ref, o_ref):
    o_ref[...] = jnp.sum(a_ref[...], axis=0)


@jax.jit
def kernel(x, batch, W, b):
    f32 = jnp.float32
    i32 = jnp.int32

    k1 = pl.pallas_call(
        _k1_body,
        grid=(NB_TC,),
        in_specs=[
            pl.BlockSpec((BT, D), lambda i: (i, 0)),
            pl.BlockSpec((1, D), lambda i: (0, 0)),
        ],
        out_specs=[
            pl.BlockSpec((BT,), lambda i: (i,)),
            pl.BlockSpec((1, 1), lambda i: (0, 0)),
        ],
        out_shape=[
            jax.ShapeDtypeStruct((NPAD,), f32),
            jax.ShapeDtypeStruct((1, 1), f32),
        ],
        scratch_shapes=[pltpu.VMEM((1, 1), f32)],
    )
    w_arr, z1 = k1(x, W.astype(f32))
    mz = jnp.broadcast_to(z1.reshape(1), (128,))
    batch_i = batch.astype(i32)

    mesh = plsc.VectorSubcoreMesh(core_axis_name="c", subcore_axis_name="s")
    params = pltpu.CompilerParams(needs_layout_passes=False)

    k2 = pl.kernel(
        _k2_body,
        out_type=jax.ShapeDtypeStruct((NW, S), f32),
        mesh=mesh,
        compiler_params=params,
        scratch_types=[
            pltpu.VMEM((WLEN,), f32),
            pltpu.VMEM((WLEN,), i32),
            pltpu.VMEM((128,), f32),
            pltpu.VMEM((S,), f32),
            pltpu.SemaphoreType.DMA,
            pltpu.SemaphoreType.DMA,
        ],
    )
    dpart = k2(w_arr, batch_i, mz)

    k3 = pl.kernel(
        _k3_body,
        out_type=jax.ShapeDtypeStruct((NW, S, D), f32),
        mesh=mesh,
        compiler_params=params,
        scratch_types=[
            pltpu.VMEM((BLK, D), f32),
            pltpu.VMEM((BLK, D), f32),
            pltpu.VMEM((WLEN,), f32),
            pltpu.VMEM((WLEN,), i32),
            pltpu.VMEM((128,), f32),
            pltpu.VMEM((NW, S), f32),
            pltpu.VMEM((S,), f32),
            pltpu.VMEM((S, D), f32),
            pltpu.SemaphoreType.DMA,
            pltpu.SemaphoreType.DMA,
            pltpu.SemaphoreType.DMA,
            pltpu.SemaphoreType.DMA,
        ],
    )
    acc = k3(x, w_arr, batch_i, mz, dpart)

    pooled = pl.pallas_call(
        _k4_body,
        out_shape=jax.ShapeDtypeStruct((S, D), f32),
    )(acc)
    return pooled


# speedup vs baseline: 1.1130x; 1.0263x over previous
import jax
import jax.numpy as jnp
from jax import lax
from jax.experimental import pallas as pl
from jax.experimental.pallas import tpu as pltpu
from jax.experimental.pallas import tpu_sc as plsc

N = 100000
D = 128
S = 64
BLK = 400
NBLK = N // BLK
NW = 32
GRP = BLK // 16
CAP = (NBLK + NW - 1) // NW
WLEN = CAP * BLK


def _wid():
    return lax.axis_index("s") * 2 + lax.axis_index("c")


def _nblk(wid):
    return (NBLK - wid + NW - 1) // NW


def _z_read(mzv):
    zero16 = jnp.zeros((16,), jnp.int32)
    zv = plsc.load_gather(mzv, [zero16])
    invzv = jnp.ones((16,), jnp.float32) / zv
    return invzv


BT = 16384
NB_TC = (N + BT - 1) // BT
NPAD = NB_TC * BT


def _k1_body(x_ref, w_ref, o_ref, zz_ref, zacc):
    i = pl.program_id(0)
    wv = lax.dot_general(w_ref[...], x_ref[...],
                         (((1,), (1,)), ((), ())),
                         preferred_element_type=jnp.float32)
    o_ref[...] = wv[0]
    col = lax.broadcasted_iota(jnp.int32, (1, BT), 1)
    ev = jnp.where(col < N - i * BT, jnp.exp(wv), 0.0)
    z_blk = jnp.sum(ev, axis=1, keepdims=True)

    @pl.when(i == 0)
    def _():
        zacc[...] = jnp.zeros((1, 1), jnp.float32)

    z_new = zacc[...] + z_blk
    zacc[...] = z_new

    @pl.when(i == NB_TC - 1)
    def _():
        zz_ref[...] = z_new


def _fire_drain_wb(w_hbm, b_hbm, wbuf, bbuf, semw, semb, wid, nblk):
    for i in range(CAP):
        @pl.when(i < nblk)
        def _():
            r0 = (wid + i * NW) * BLK
            pltpu.async_copy(w_hbm.at[pl.ds(r0, BLK)],
                             wbuf.at[pl.ds(i * BLK, BLK)], semw)
            pltpu.async_copy(b_hbm.at[pl.ds(r0, BLK)],
                             bbuf.at[pl.ds(i * BLK, BLK)], semb)
    for i in range(CAP):
        @pl.when(i < nblk)
        def _():
            pltpu.make_async_copy(w_hbm.at[pl.ds(0, BLK)],
                                  wbuf.at[pl.ds(0, BLK)], semw).wait()
            pltpu.make_async_copy(b_hbm.at[pl.ds(0, BLK)],
                                  bbuf.at[pl.ds(0, BLK)], semb).wait()


def _k2_body(w_hbm, b_hbm, mz_hbm, dp_hbm, wbuf, bbuf, mzv, denomv,
             semw, semb):
    wid = _wid()
    nblk = _nblk(wid)
    _fire_drain_wb(w_hbm, b_hbm, wbuf, bbuf, semw, semb, wid, nblk)
    pltpu.sync_copy(mz_hbm, mzv)
    invzv = _z_read(mzv)
    zero16 = jnp.zeros((16,), jnp.float32)
    for c in range(S // 16):
        denomv[pl.ds(16 * c, 16)] = zero16

    def grp_body(g, _):
        wvec = wbuf[pl.ds(16 * g, 16)]
        ev = jnp.exp(jnp.exp(wvec) * invzv)
        plsc.addupdate_scatter(denomv, [bbuf[pl.ds(16 * g, 16)]], ev)
        return 0

    lax.fori_loop(0, nblk * GRP, grp_body, 0)
    pltpu.sync_copy(denomv, dp_hbm.at[wid])


def _k3_body(x_hbm, w_hbm, b_hbm, mz_hbm, dp_hbm, acc_hbm,
             xv0, xv1, wbuf, bbuf, mzv, dpv, cinvv, accv,
             sem0, sem1, semw, semb):
    wid = _wid()
    nblk = _nblk(wid)
    pltpu.async_copy(x_hbm.at[pl.ds(wid * BLK, BLK)], xv0, sem0)
    _fire_drain_wb(w_hbm, b_hbm, wbuf, bbuf, semw, semb, wid, nblk)
    pltpu.sync_copy(mz_hbm, mzv)
    invzv = _z_read(mzv)
    pltpu.sync_copy(dp_hbm, dpv)
    one16 = jnp.ones((16,), jnp.float32)
    for c in range(S // 16):
        s = dpv[0, pl.ds(16 * c, 16)]
        for r in range(1, NW):
            s = s + dpv[r, pl.ds(16 * c, 16)]
        cinvv[pl.ds(16 * c, 16)] = one16 / s

    zero16 = jnp.zeros((16,), jnp.float32)

    def zero_body(r, _):
        for j in range(D // 16):
            accv[r, pl.ds(16 * j, 16)] = zero16
        return 0

    lax.fori_loop(0, S, zero_body, 0)

    def blk_body(i, _):
        even = (i % 2) == 0
        nxt = i + 1

        @pl.when(jnp.logical_and(nxt < nblk, even))
        def _():
            pltpu.async_copy(
                x_hbm.at[pl.ds((wid + nxt * NW) * BLK, BLK)], xv1, sem1)

        @pl.when(jnp.logical_and(nxt < nblk, jnp.logical_not(even)))
        def _():
            pltpu.async_copy(
                x_hbm.at[pl.ds((wid + nxt * NW) * BLK, BLK)], xv0, sem0)

        def mk(xv, sem):
            def go():
                pltpu.make_async_copy(x_hbm.at[pl.ds(0, BLK)], xv, sem).wait()

                def grp_body(g, _):
                    wvec = wbuf[pl.ds(i * BLK + 16 * g, 16)]
                    bvec = bbuf[pl.ds(i * BLK + 16 * g, 16)]
                    gv = jnp.exp(wvec) * invzv
                    nw = jnp.exp(gv) * plsc.load_gather(cinvv, [bvec])

                    def uniform():
                        accs = [jnp.zeros((16,), jnp.float32)
                                for _ in range(D // 16)]
                        for l in range(16):
                            r = 16 * g + l
                            sv = jnp.full((16,), nw[l], jnp.float32)
                            for j in range(D // 16):
                                accs[j] = accs[j] + xv[r, pl.ds(16 * j, 16)] * sv
                        bi = bvec[0]
                        for j in range(D // 16):
                            plsc.addupdate(accv.at[bi, pl.ds(16 * j, 16)],
                                           accs[j])

                    def mixed():
                        for l in range(16):
                            bi = bvec[l]
                            sv = jnp.full((16,), nw[l], jnp.float32)
                            r = 16 * g + l
                            for j in range(D // 16):
                                plsc.addupdate(
                                    accv.at[bi, pl.ds(16 * j, 16)],
                                    xv[r, pl.ds(16 * j, 16)] * sv)

                    lax.cond(bvec[0] == bvec[15], uniform, mixed)
                    return 0

                return lax.fori_loop(0, GRP, grp_body, 0)
            return go

        return lax.cond(even, mk(xv0, sem0), mk(xv1, sem1))

    lax.fori_loop(0, nblk, blk_body, 0)
    pltpu.sync_copy(accv, acc_hbm.at[wid])


def _k4_body(a_ref, o_ref):
    o_ref[...] = jnp.sum(a_ref[...], axis=0)


@jax.jit
def kernel(x, batch, W, b):
    f32 = jnp.float32
    i32 = jnp.int32

    k1 = pl.pallas_call(
        _k1_body,
        grid=(NB_TC,),
        in_specs=[
            pl.BlockSpec((BT, D), lambda i: (i, 0)),
            pl.BlockSpec((1, D), lambda i: (0, 0)),
        ],
        out_specs=[
            pl.BlockSpec((BT,), lambda i: (i,)),
            pl.BlockSpec((1, 1), lambda i: (0, 0)),
        ],
        out_shape=[
            jax.ShapeDtypeStruct((NPAD,), f32),
            jax.ShapeDtypeStruct((1, 1), f32),
        ],
        scratch_shapes=[pltpu.VMEM((1, 1), f32)],
    )
    w_arr, z1 = k1(x, W.astype(f32))
    mz = jnp.broadcast_to(z1.reshape(1), (128,))
    batch_i = batch.astype(i32)

    mesh = plsc.VectorSubcoreMesh(core_axis_name="c", subcore_axis_name="s")
    params = pltpu.CompilerParams(needs_layout_passes=False)

    k2 = pl.kernel(
        _k2_body,
        out_type=jax.ShapeDtypeStruct((NW, S), f32),
        mesh=mesh,
        compiler_params=params,
        scratch_types=[
            pltpu.VMEM((WLEN,), f32),
            pltpu.VMEM((WLEN,), i32),
            pltpu.VMEM((128,), f32),
            pltpu.VMEM((S,), f32),
            pltpu.SemaphoreType.DMA,
            pltpu.SemaphoreType.DMA,
        ],
    )
    dpart = k2(w_arr, batch_i, mz)

    k3 = pl.kernel(
        _k3_body,
        out_type=jax.ShapeDtypeStruct((NW, S, D), f32),
        mesh=mesh,
        compiler_params=params,
        scratch_types=[
            pltpu.VMEM((BLK, D), f32),
            pltpu.VMEM((BLK, D), f32),
            pltpu.VMEM((WLEN,), f32),
            pltpu.VMEM((WLEN,), i32),
            pltpu.VMEM((128,), f32),
            pltpu.VMEM((NW, S), f32),
            pltpu.VMEM((S,), f32),
            pltpu.VMEM((S, D), f32),
            pltpu.SemaphoreType.DMA,
            pltpu.SemaphoreType.DMA,
            pltpu.SemaphoreType.DMA,
            pltpu.SemaphoreType.DMA,
        ],
    )
    acc = k3(x, w_arr, batch_i, mz, dpart)

    pooled = pl.pallas_call(
        _k4_body,
        out_shape=jax.ShapeDtypeStruct((S, D), f32),
    )(acc)
    return pooled
